# baseline (speedup 1.0000x reference)
import jax
import jax.numpy as jnp
from jax.experimental import pallas as pl

_FILL = 1e-15
_N = 32
_CH = 256


def _max_layer_kernel(x_ref, o_ref):
    x = x_ref[0]
    R, L = x.shape
    G = L // _N
    TOP = _N * _N // L

    ch = min(_CH, R)
    lane = jax.lax.broadcasted_iota(jnp.int32, (ch, L), 1)
    sub = lane & (_N - 1)
    base = sub == 0

    acc = jnp.zeros((8, L), jnp.int32)
    for c in range(R // ch):
        xc = x[c * ch:(c + 1) * ch]
        v = xc
        for s in (16, 8, 4, 2, 1):
            v = jnp.maximum(v, jnp.roll(v, -s, axis=1))
        for s in (1, 2, 4, 8, 16):
            v = jnp.where((sub & s) != 0, jnp.roll(v, s, axis=1), v)
        mi = jnp.where(xc == v, sub, _N)
        for s in (16, 8, 4, 2, 1):
            mi = jnp.minimum(mi, jnp.roll(mi, -s, axis=1))
        contrib = jnp.where(base, jnp.left_shift(jnp.int32(1), mi), 0)
        h = ch
        while h > 8:
            h //= 2
            contrib = contrib[:h] | contrib[h:2 * h]
        acc = acc | contrib

    for s in (4, 2, 1):
        acc = acc | jnp.roll(acc, s, axis=0)
    m = acc[:1]
    for s in (64, 32, 16, 8, 4, 2, 1):
        m = m | jnp.roll(m, s, axis=1)

    qi = jax.lax.broadcasted_iota(jnp.int32, (TOP, L), 0)
    ci = jax.lax.broadcasted_iota(jnp.int32, (TOP, L), 1)
    rn = G * qi + ci // _N
    keep = (jnp.right_shift(jnp.broadcast_to(m, (TOP, L)), rn) & 1) == 1

    o_ref[0] = jnp.full((R, L), _FILL, jnp.float32)
    o_ref[0, :TOP, :] = jnp.where(keep, x[:TOP, :], jnp.full((TOP, L), _FILL, jnp.float32))


@jax.jit
def kernel(X):
    B, M, N = X.shape
    R = M * N // 128
    Xv = X.reshape(B, R, 128)
    out = pl.pallas_call(
        _max_layer_kernel,
        grid=(B,),
        in_specs=[pl.BlockSpec((1, R, 128), lambda i: (i, 0, 0))],
        out_specs=pl.BlockSpec((1, R, 128), lambda i: (i, 0, 0)),
        out_shape=jax.ShapeDtypeStruct((B, R, 128), jnp.float32),
    )(Xv)
    return out.reshape(B, M, N)


# speedup vs baseline: 1.0750x; 1.0750x over previous
import functools
import jax
import jax.numpy as jnp
from jax import lax
from jax.experimental import pallas as pl
from jax.experimental.pallas import tpu as pltpu
from jax.experimental.pallas import tpu_sc as plsc

_FILL = 1e-15
_N = 32
_NC = 2
_NS = 16
_NW = _NC * _NS
_TILE = 512


def _top_kernel(x_ref, o_ref):
    x = x_ref[0]
    R, L = x.shape
    G = L // _N
    TOP = _N * _N // L

    lane = jax.lax.broadcasted_iota(jnp.int32, (R, L), 1)
    sub = lane & (_N - 1)

    v = x
    for s in (16, 8, 4, 2, 1):
        v = jnp.maximum(v, jnp.roll(v, -s, axis=1))
    for s in (1, 2, 4, 8, 16):
        v = jnp.where((sub & s) != 0, jnp.roll(v, s, axis=1), v)

    mi = jnp.where(x == v, sub, _N)
    for s in (16, 8, 4, 2, 1):
        mi = jnp.minimum(mi, jnp.roll(mi, -s, axis=1))
    for s in (1, 2, 4, 8, 16):
        mi = jnp.where((sub & s) != 0, jnp.roll(mi, s, axis=1), mi)

    oh = (mi == sub).astype(jnp.int32)
    red = jnp.max(oh, axis=0, keepdims=True)
    red = jnp.maximum(red, jnp.roll(red, 64, axis=1))
    red = jnp.maximum(red, jnp.roll(red, 32, axis=1))

    qi = jax.lax.broadcasted_iota(jnp.int32, (TOP, L), 0)
    ci = jax.lax.broadcasted_iota(jnp.int32, (TOP, L), 1)
    a = jnp.where(ci // G == qi, jnp.broadcast_to(red.astype(jnp.float32), (TOP, L)), 0.0)
    ri = jax.lax.broadcasted_iota(jnp.int32, (L, L), 0)
    li = jax.lax.broadcasted_iota(jnp.int32, (L, L), 1)
    b = jnp.where(ri % G == li // _N, 1.0, 0.0).astype(jnp.float32)
    keep = (
        jax.lax.dot_general(a, b, (((1,), (0,)), ((), ())),
                            preferred_element_type=jnp.float32)
        > 0.5
    )

    o_ref[0] = jnp.where(keep, x[:TOP, :], jnp.full((TOP, L), _FILL, jnp.float32))


def _dus_kernel(t_ref, f_ref, o_ref):
    o_ref[0] = t_ref[0]


def _make_fill(total_rows):
    rows_per_w = total_rows // _NW
    n_copies = rows_per_w // _TILE
    mesh = plsc.VectorSubcoreMesh(core_axis_name="c", subcore_axis_name="s")

    @functools.partial(
        pl.kernel,
        mesh=mesh,
        out_type=jax.ShapeDtypeStruct((total_rows, 128), jnp.float32),
        scratch_types=[
            pltpu.VMEM((_TILE, 128), jnp.float32),
            pltpu.SemaphoreType.DMA,
        ],
    )
    def fill_k(tile_hbm, out_hbm, tile_v, sem):
        wid = lax.axis_index("s") * _NC + lax.axis_index("c")
        base = wid * rows_per_w
        pltpu.sync_copy(tile_hbm, tile_v)
        handles = []
        for i in range(n_copies):
            handles.append(
                pltpu.async_copy(tile_v, out_hbm.at[pl.ds(base + i * _TILE, _TILE)], sem)
            )
        for h in handles:
            h.wait()

    return fill_k


@jax.jit
def kernel(X):
    B, M, N = X.shape
    R = M * N // 128
    TOP = N * N // 128
    Xv = X.reshape(B, R, 128)

    tops = pl.pallas_call(
        _top_kernel,
        grid=(B,),
        in_specs=[pl.BlockSpec((1, R, 128), lambda i: (i, 0, 0))],
        out_specs=pl.BlockSpec((1, TOP, 128), lambda i: (i, 0, 0)),
        out_shape=jax.ShapeDtypeStruct((B, TOP, 128), jnp.float32),
    )(Xv)

    tile = jnp.full((_TILE, 128), _FILL, jnp.float32)
    filled = _make_fill(B * R)(tile).reshape(B, R, 128)

    out = pl.pallas_call(
        _dus_kernel,
        grid=(B,),
        in_specs=[
            pl.BlockSpec((1, TOP, 128), lambda i: (i, 0, 0)),
            pl.BlockSpec((1, TOP, 128), lambda i: (i, 0, 0)),
        ],
        out_specs=pl.BlockSpec((1, TOP, 128), lambda i: (i, 0, 0)),
        out_shape=jax.ShapeDtypeStruct((B, R, 128), jnp.float32),
        input_output_aliases={1: 0},
    )(tops, filled)
    return out.reshape(B, M, N)
